# baseline (speedup 1.0000x reference)
import functools

import jax
import jax.numpy as jnp
from jax import lax
from jax.experimental import pallas as pl
from jax.experimental.pallas import tpu as pltpu
from jax.experimental.pallas import tpu_sc as plsc

L = 16
NC, NS = 2, 16
NW = NC * NS
ROWS, COLS = 16384, 200
RPT = ROWS // NW
NCHUNK = 4
NXBUF = 2
RCH = RPT // NCHUNK
CSTARTS = list(range(0, 192, 16)) + [184]


@functools.partial(
    pl.kernel,
    out_type=jax.ShapeDtypeStruct((ROWS, COLS), jnp.int16),
    mesh=plsc.VectorSubcoreMesh(core_axis_name="c", subcore_axis_name="s"),
    compiler_params=pltpu.CompilerParams(
        needs_layout_passes=False, use_tc_tiling_on_sc=True),
    scratch_types=[
        pltpu.VMEM((256,), jnp.int32),
        pltpu.VMEM((RCH, COLS), jnp.int32),
        pltpu.VMEM((RCH, COLS), jnp.int32),
        pltpu.VMEM((RCH // 2, COLS), jnp.int32),
        pltpu.VMEM((RCH // 2, COLS), jnp.int32),
        pltpu.SemaphoreType.DMA,
        pltpu.SemaphoreType.DMA,
        pltpu.SemaphoreType.DMA,
        pltpu.SemaphoreType.DMA,
    ],
)
def _lut_kernel(col_hbm, a_hbm, out_hbm,
                col_v, x0_v, x1_v, o0_v, o1_v,
                is0, is1, os0, os1):
    wid = lax.axis_index("s") * NC + lax.axis_index("c")
    rbase = wid * RPT
    x_bufs = (x0_v, x1_v, x0_v, x1_v)
    o_bufs = (o0_v, o1_v)
    i_sems = (is0, is1, is0, is1)
    o_sems = (os0, os1)

    in_cp = {}
    for c in range(NXBUF):
        in_cp[c] = pltpu.async_copy(
            a_hbm.at[pl.ds(rbase + c * RCH, RCH), :], x_bufs[c], i_sems[c])
    pltpu.sync_copy(col_hbm, col_v)

    iota = lax.iota(jnp.int32, L)

    def compute_chunk(x_ref, o_ref):
        @plsc.parallel_loop(0, RCH // 2, unroll=3)
        def body(q):
            for c0 in CSTARTS:
                av_e = x_ref[2 * q, pl.ds(c0, L)]
                av_o = x_ref[2 * q + 1, pl.ds(c0, L)]
                ge = plsc.load_gather(col_v, [av_e + 128])
                go = plsc.load_gather(col_v, [av_o + 128])
                w = lax.bitwise_or(
                    lax.bitwise_and(ge, 0xFFFF), lax.shift_left(go, 16))
                o_ref[q, pl.ds(c0, L)] = w

    out_cp = {}
    for c in range(NCHUNK):
        p = c & 1
        in_cp[c].wait()
        if c >= 2:
            out_cp[c - 2].wait()
        compute_chunk(x_bufs[c], o_bufs[p])
        out_cp[c] = pltpu.async_copy(
            o_bufs[p].bitcast(jnp.int16),
            out_hbm.at[pl.ds(rbase + c * RCH, RCH), :], o_sems[p])
        if c + NXBUF < NCHUNK:
            in_cp[c + NXBUF] = pltpu.async_copy(
                a_hbm.at[pl.ds(rbase + (c + NXBUF) * RCH, RCH), :],
                x_bufs[c + NXBUF], i_sems[c + NXBUF])
    out_cp[NCHUNK - 2].wait()
    out_cp[NCHUNK - 1].wait()


def kernel(a, b, table):
    idx_b = jnp.asarray(b, jnp.int32) + 128
    column = lax.dynamic_slice_in_dim(table, idx_b, 1, axis=1)
    col_i32 = column.reshape(256).astype(jnp.int32)
    return _lut_kernel(col_i32, a.astype(jnp.int32))


# speedup vs baseline: 1.0293x; 1.0293x over previous
import functools

import jax
import jax.numpy as jnp
from jax import lax
from jax.experimental import pallas as pl
from jax.experimental.pallas import tpu as pltpu
from jax.experimental.pallas import tpu_sc as plsc

L = 16
NC, NS = 2, 16
NW = NC * NS
ROWS, COLS = 16384, 200
RPT = ROWS // NW
NCHUNK = 4
NXBUF = 2
RCH = RPT // NCHUNK
CSTARTS = list(range(0, 192, 32)) + [168]


@functools.partial(
    pl.kernel,
    out_type=jax.ShapeDtypeStruct((ROWS, COLS), jnp.int16),
    mesh=plsc.VectorSubcoreMesh(core_axis_name="c", subcore_axis_name="s"),
    compiler_params=pltpu.CompilerParams(
        needs_layout_passes=False, use_tc_tiling_on_sc=True),
    scratch_types=[
        pltpu.VMEM((256,), jnp.int32),
        pltpu.VMEM((RCH, COLS), jnp.int32),
        pltpu.VMEM((RCH, COLS), jnp.int32),
        pltpu.VMEM((RCH // 2, COLS), jnp.int32),
        pltpu.VMEM((RCH // 2, COLS), jnp.int32),
        pltpu.SemaphoreType.DMA,
        pltpu.SemaphoreType.DMA,
        pltpu.SemaphoreType.DMA,
        pltpu.SemaphoreType.DMA,
    ],
)
def _lut_kernel(col_hbm, a_hbm, out_hbm,
                col_v, x0_v, x1_v, o0_v, o1_v,
                is0, is1, os0, os1):
    wid = lax.axis_index("s") * NC + lax.axis_index("c")
    rbase = wid * RPT
    x_bufs = (x0_v, x1_v, x0_v, x1_v)
    o_bufs = (o0_v, o1_v)
    i_sems = (is0, is1, is0, is1)
    o_sems = (os0, os1)

    in_cp = {}
    for c in range(NXBUF):
        in_cp[c] = pltpu.async_copy(
            a_hbm.at[pl.ds(rbase + c * RCH, RCH), :], x_bufs[c], i_sems[c])
    pltpu.sync_copy(col_hbm, col_v)

    iota = lax.iota(jnp.int32, L)
    iota2 = iota * 2

    def compute_chunk(x_ref, o_ref):
        xv16 = x_ref.bitcast(jnp.int16)

        @plsc.parallel_loop(0, RCH // 2, unroll=2)
        def body(q):
            qv = jnp.full((L,), q, jnp.int32)
            for c0 in CSTARTS:
                a_top = xv16[4 * q, pl.ds(c0, 2 * L)]
                a_bot = xv16[4 * q + 2, pl.ds(c0, 2 * L)]
                te, to = plsc.unpack(a_top, format=plsc.PackFormat.INTERLEAVED)
                be, bo = plsc.unpack(a_bot, format=plsc.PackFormat.INTERLEAVED)
                ge_t = plsc.load_gather(col_v, [te + 128])
                go_t = plsc.load_gather(col_v, [to + 128])
                ge_b = plsc.load_gather(col_v, [be + 128])
                go_b = plsc.load_gather(col_v, [bo + 128])
                w_e = lax.bitwise_or(
                    lax.bitwise_and(ge_t, 0xFFFF), lax.shift_left(ge_b, 16))
                w_o = lax.bitwise_or(
                    lax.bitwise_and(go_t, 0xFFFF), lax.shift_left(go_b, 16))
                plsc.store_scatter(o_ref, [qv, c0 + iota2], w_e)
                plsc.store_scatter(o_ref, [qv, c0 + iota2 + 1], w_o)

    out_cp = {}
    for c in range(NCHUNK):
        p = c & 1
        in_cp[c].wait()
        if c >= 2:
            out_cp[c - 2].wait()
        compute_chunk(x_bufs[c], o_bufs[p])
        out_cp[c] = pltpu.async_copy(
            o_bufs[p].bitcast(jnp.int16),
            out_hbm.at[pl.ds(rbase + c * RCH, RCH), :], o_sems[p])
        if c + NXBUF < NCHUNK:
            in_cp[c + NXBUF] = pltpu.async_copy(
                a_hbm.at[pl.ds(rbase + (c + NXBUF) * RCH, RCH), :],
                x_bufs[c + NXBUF], i_sems[c + NXBUF])
    out_cp[NCHUNK - 2].wait()
    out_cp[NCHUNK - 1].wait()


def kernel(a, b, table):
    idx_b = jnp.asarray(b, jnp.int32) + 128
    column = lax.dynamic_slice_in_dim(table, idx_b, 1, axis=1)
    col_i32 = column.reshape(256).astype(jnp.int32)
    return _lut_kernel(col_i32, a.astype(jnp.int32))
